# baseline (speedup 1.0000x reference)
import functools

import jax
import jax.numpy as jnp
from jax import lax
from jax.experimental import pallas as pl
from jax.experimental.pallas import tpu as pltpu
from jax.experimental.pallas import tpu_sc as plsc

N = 10000
E = 320000
IN_DIM = 128
HID = 64
ALIGN = 32
NUM_CLASSES = 7

NC = 2
NS = 16
NW = NC * NS
CH = 128
NCHUNK = E // CH
BASE = NCHUNK // NW
EXTRA = NCHUNK - BASE * NW
CMAX = BASE + 1
SROWS = 10240
RPT = SROWS // NS

BR = 2048
GRID = (N + BR - 1) // BR

_mesh = plsc.VectorSubcoreMesh(core_axis_name="c", subcore_axis_name="s")
_sc_params = pltpu.CompilerParams(use_tc_tiling_on_sc=False)


def _worker_bounds(cid, sid):
    w = cid * NS + sid
    row_start = w * BASE + jnp.minimum(w, EXTRA)
    ncap = BASE + jnp.where(w < EXTRA, 1, 0)
    return w, row_start, ncap


def _dst_slab(dst_hbm, dst_v, row_start, w):
    pltpu.sync_copy(dst_hbm.at[pl.ds(row_start * CH, BASE * CH)],
                    dst_v.at[pl.ds(0, BASE * CH)])

    @pl.when(w < EXTRA)
    def _():
        pltpu.sync_copy(dst_hbm.at[pl.ds((row_start + BASE) * CH, CH)],
                        dst_v.at[pl.ds(BASE * CH, CH)])


@functools.partial(
    pl.kernel,
    out_type=jax.ShapeDtypeStruct((NC, SROWS, 16), jnp.float32),
    mesh=_mesh,
    scratch_types=[
        pltpu.VMEM((CMAX * CH,), jnp.int32),
        pltpu.VMEM((CH, 16), jnp.float32),
        pltpu.VMEM_SHARED((SROWS, 16), jnp.float32),
    ],
    compiler_params=_sc_params,
)
def _deg_kernel(dst_hbm, zeros_hbm, ones_hbm, deg_out, dst_v, ones_v, acc_sh):
    cid = lax.axis_index("c")
    sid = lax.axis_index("s")
    w, row_start, ncap = _worker_bounds(cid, sid)
    _dst_slab(dst_hbm, dst_v, row_start, w)
    pltpu.sync_copy(ones_hbm, ones_v)

    pltpu.sync_copy(zeros_hbm, acc_sh.at[pl.ds(sid * RPT, RPT)])
    plsc.subcore_barrier()

    def chunk(j, _):
        pltpu.sync_copy(ones_v, acc_sh.at[dst_v.at[pl.ds(j * CH, CH)]], add=True)
        return ()

    lax.fori_loop(0, ncap, chunk, ())
    plsc.subcore_barrier()
    pltpu.sync_copy(acc_sh.at[pl.ds(sid * RPT, RPT)],
                    deg_out.at[cid, pl.ds(sid * RPT, RPT)])


@functools.partial(
    pl.kernel,
    out_type=jax.ShapeDtypeStruct((NC, SROWS, HID), jnp.float32),
    mesh=_mesh,
    scratch_types=[
        pltpu.VMEM((CMAX * CH,), jnp.int32),
        pltpu.VMEM((CMAX * CH,), jnp.int32),
        pltpu.VMEM((CH, HID), jnp.float32),
        pltpu.VMEM((CH, HID), jnp.float32),
        pltpu.SemaphoreType.DMA,
        pltpu.SemaphoreType.DMA,
        pltpu.VMEM_SHARED((SROWS, HID), jnp.float32),
        pltpu.VMEM_SHARED((SROWS, HID), jnp.float32),
    ],
    compiler_params=_sc_params,
)
def _msg_kernel(src_hbm, dst_hbm, y_hbm, zeros_hbm, s_out,
                src_v, dst_v, buf0, buf1, sem0, sem1, acc_sh, y_sh):
    cid = lax.axis_index("c")
    sid = lax.axis_index("s")
    w, row_start, ncap = _worker_bounds(cid, sid)
    pltpu.sync_copy(src_hbm.at[pl.ds(row_start * CH, BASE * CH)],
                    src_v.at[pl.ds(0, BASE * CH)])

    @pl.when(w < EXTRA)
    def _():
        pltpu.sync_copy(src_hbm.at[pl.ds((row_start + BASE) * CH, CH)],
                        src_v.at[pl.ds(BASE * CH, CH)])

    _dst_slab(dst_hbm, dst_v, row_start, w)

    pltpu.sync_copy(y_hbm.at[pl.ds(sid * RPT, RPT)],
                    y_sh.at[pl.ds(sid * RPT, RPT)])
    pltpu.sync_copy(zeros_hbm, acc_sh.at[pl.ds(sid * RPT, RPT)])
    plsc.subcore_barrier()

    def src_idx(j):
        return src_v.at[pl.ds(j * CH, CH)]

    pltpu.async_copy(y_sh.at[src_idx(0)], buf0, sem0)
    pltpu.async_copy(y_sh.at[src_idx(1)], buf1, sem1)

    def pair(i, _):
        j0 = i * 2
        for b, (buf, sem) in enumerate(((buf0, sem0), (buf1, sem1))):
            j = j0 + b
            pltpu.make_async_copy(y_sh.at[src_idx(j)], buf, sem).wait()
            pltpu.sync_copy(buf, acc_sh.at[dst_v.at[pl.ds(j * CH, CH)]], add=True)

            @pl.when(j + 2 < ncap)
            def _():
                pltpu.async_copy(y_sh.at[src_idx(j + 2)], buf, sem)

        return ()

    lax.fori_loop(0, BASE // 2, pair, ())

    @pl.when(ncap > BASE)
    def _():
        pltpu.make_async_copy(y_sh.at[src_idx(BASE)], buf0, sem0).wait()
        pltpu.sync_copy(buf0, acc_sh.at[dst_v.at[pl.ds(BASE * CH, CH)]], add=True)

    plsc.subcore_barrier()
    pltpu.sync_copy(acc_sh.at[pl.ds(sid * RPT, RPT)],
                    s_out.at[cid, pl.ds(sid * RPT, RPT)])


def _proj_body(x_ref, wg_ref, wps_ref, bps_ref, xw_ref, zsem_ref):
    x = x_ref[...]
    xw_ref[...] = jnp.dot(x, wg_ref[...], preferred_element_type=jnp.float32)
    zsem_ref[...] = (
        jnp.dot(x, wps_ref[...], preferred_element_type=jnp.float32)
        + bps_ref[...]
    )


def _proj(x, W_gcn, W_ps, b_ps):
    return pl.pallas_call(
        _proj_body,
        grid=(GRID,),
        in_specs=[
            pl.BlockSpec((BR, IN_DIM), lambda i: (i, 0)),
            pl.BlockSpec((IN_DIM, HID), lambda i: (0, 0)),
            pl.BlockSpec((IN_DIM, ALIGN), lambda i: (0, 0)),
            pl.BlockSpec((1, ALIGN), lambda i: (0, 0)),
        ],
        out_specs=[
            pl.BlockSpec((BR, HID), lambda i: (i, 0)),
            pl.BlockSpec((BR, ALIGN), lambda i: (i, 0)),
        ],
        out_shape=[
            jax.ShapeDtypeStruct((N, HID), jnp.float32),
            jax.ShapeDtypeStruct((N, ALIGN), jnp.float32),
        ],
    )(x, W_gcn, W_ps, b_ps)


def _scale_body(dp_ref, xw_ref, y_ref, dinv_ref):
    deg = dp_ref[0, :, 0:1] + dp_ref[1, :, 0:1] + 1.0
    dinv = lax.rsqrt(deg)
    y_ref[...] = dinv * xw_ref[...]
    dinv_ref[...] = jnp.broadcast_to(dinv, dinv_ref.shape)


def _scale(deg_parts, xw):
    return pl.pallas_call(
        _scale_body,
        grid=(GRID,),
        in_specs=[
            pl.BlockSpec((2, BR, 16), lambda i: (0, i, 0)),
            pl.BlockSpec((BR, HID), lambda i: (i, 0)),
        ],
        out_specs=[
            pl.BlockSpec((BR, HID), lambda i: (i, 0)),
            pl.BlockSpec((BR, 16), lambda i: (i, 0)),
        ],
        out_shape=[
            jax.ShapeDtypeStruct((SROWS, HID), jnp.float32),
            jax.ShapeDtypeStruct((N, 16), jnp.float32),
        ],
    )(deg_parts, xw)


def _head_body(s_ref, y_ref, dinv_ref, zsem_ref, bg_ref, wpt_ref, bpt_ref,
               wcls_ref, bcls_ref, logits_ref, an_ref, ztopo_ref):
    dinv = dinv_ref[:, 0:1]
    s_tot = s_ref[0] + s_ref[1] + y_ref[...]
    h = jnp.maximum(dinv * s_tot + bg_ref[...], 0.0)
    z_topo = (
        jnp.dot(h, wpt_ref[...], preferred_element_type=jnp.float32)
        + bpt_ref[...]
    )
    logits_ref[...] = (
        jnp.dot(z_topo, wcls_ref[...], preferred_element_type=jnp.float32)
        + bcls_ref[...]
    )
    diff = z_topo - zsem_ref[...]
    an_ref[...] = jnp.sqrt(jnp.sum(diff * diff, axis=1))
    ztopo_ref[...] = z_topo


def _heads(s_parts, y, dinv, z_sem, b_gcn, W_pt, b_pt, W_cls, b_cls):
    return pl.pallas_call(
        _head_body,
        grid=(GRID,),
        in_specs=[
            pl.BlockSpec((2, BR, HID), lambda i: (0, i, 0)),
            pl.BlockSpec((BR, HID), lambda i: (i, 0)),
            pl.BlockSpec((BR, 16), lambda i: (i, 0)),
            pl.BlockSpec((BR, ALIGN), lambda i: (i, 0)),
            pl.BlockSpec((1, HID), lambda i: (0, 0)),
            pl.BlockSpec((HID, ALIGN), lambda i: (0, 0)),
            pl.BlockSpec((1, ALIGN), lambda i: (0, 0)),
            pl.BlockSpec((ALIGN, NUM_CLASSES), lambda i: (0, 0)),
            pl.BlockSpec((1, NUM_CLASSES), lambda i: (0, 0)),
        ],
        out_specs=[
            pl.BlockSpec((BR, NUM_CLASSES), lambda i: (i, 0)),
            pl.BlockSpec((BR,), lambda i: (i,)),
            pl.BlockSpec((BR, ALIGN), lambda i: (i, 0)),
        ],
        out_shape=[
            jax.ShapeDtypeStruct((N, NUM_CLASSES), jnp.float32),
            jax.ShapeDtypeStruct((N,), jnp.float32),
            jax.ShapeDtypeStruct((N, ALIGN), jnp.float32),
        ],
    )(s_parts, y, dinv, z_sem, b_gcn, W_pt, b_pt, W_cls, b_cls)


def kernel(x, edge_index, W_gcn, b_gcn, W_pt, b_pt, W_ps, b_ps, W_cls, b_cls):
    src_r = edge_index[0]
    dst_r = edge_index[1]

    zeros16 = jnp.zeros((RPT, 16), jnp.float32)
    zeros64 = jnp.zeros((RPT, HID), jnp.float32)
    ones16 = jnp.ones((CH, 16), jnp.float32)

    deg_parts = _deg_kernel(dst_r, zeros16, ones16)
    xw, z_sem = _proj(x, W_gcn, W_ps, b_ps.reshape(1, ALIGN))
    y, dinv = _scale(deg_parts, xw)
    s_parts = _msg_kernel(src_r, dst_r, y, zeros64)
    logits, anomaly, z_topo = _heads(
        s_parts, y, dinv, z_sem, b_gcn.reshape(1, HID), W_pt,
        b_pt.reshape(1, ALIGN), W_cls, b_cls.reshape(1, NUM_CLASSES))
    return (logits, anomaly, z_topo, z_sem)


# speedup vs baseline: 1.0581x; 1.0581x over previous
import functools

import jax
import jax.numpy as jnp
from jax import lax
from jax.experimental import pallas as pl
from jax.experimental.pallas import tpu as pltpu
from jax.experimental.pallas import tpu_sc as plsc

N = 10000
E = 320000
IN_DIM = 128
HID = 64
ALIGN = 32
NUM_CLASSES = 7

NC = 2
NS = 16
NW = NC * NS
CH = 128
NCHUNK = E // CH
BASE = NCHUNK // NW
EXTRA = NCHUNK - BASE * NW
CMAX = BASE + 1
SROWS = 10240
RPT = SROWS // NS

BR = 2048
GRID = (N + BR - 1) // BR

_mesh = plsc.VectorSubcoreMesh(core_axis_name="c", subcore_axis_name="s")
_sc_params = pltpu.CompilerParams(use_tc_tiling_on_sc=False)


def _worker_bounds(cid, sid):
    w = cid * NS + sid
    row_start = w * BASE + jnp.minimum(w, EXTRA)
    ncap = BASE + jnp.where(w < EXTRA, 1, 0)
    return w, row_start, ncap


def _dst_slab(dst_hbm, dst_v, row_start, w):
    pltpu.sync_copy(dst_hbm.at[pl.ds(row_start * CH, BASE * CH)],
                    dst_v.at[pl.ds(0, BASE * CH)])

    @pl.when(w < EXTRA)
    def _():
        pltpu.sync_copy(dst_hbm.at[pl.ds((row_start + BASE) * CH, CH)],
                        dst_v.at[pl.ds(BASE * CH, CH)])


@functools.partial(
    pl.kernel,
    out_type=jax.ShapeDtypeStruct((NC, SROWS, 16), jnp.float32),
    mesh=_mesh,
    scratch_types=[
        pltpu.VMEM((CMAX * CH,), jnp.int32),
        pltpu.VMEM((CH, 16), jnp.float32),
        pltpu.VMEM_SHARED((SROWS, 16), jnp.float32),
    ],
    compiler_params=_sc_params,
)
def _deg_kernel(ei_hbm, zeros_hbm, ones_hbm, deg_out, dst_v, ones_v, acc_sh):
    cid = lax.axis_index("c")
    sid = lax.axis_index("s")
    w, row_start, ncap = _worker_bounds(cid, sid)
    _dst_slab(ei_hbm.at[1], dst_v, row_start, w)
    pltpu.sync_copy(ones_hbm, ones_v)

    pltpu.sync_copy(zeros_hbm, acc_sh.at[pl.ds(sid * RPT, RPT)])
    plsc.subcore_barrier()

    def chunk(j, _):
        pltpu.sync_copy(ones_v, acc_sh.at[dst_v.at[pl.ds(j * CH, CH)]], add=True)
        return ()

    lax.fori_loop(0, ncap, chunk, ())
    plsc.subcore_barrier()
    pltpu.sync_copy(acc_sh.at[pl.ds(sid * RPT, RPT)],
                    deg_out.at[cid, pl.ds(sid * RPT, RPT)])


@functools.partial(
    pl.kernel,
    out_type=jax.ShapeDtypeStruct((NC, SROWS, HID), jnp.float32),
    mesh=_mesh,
    scratch_types=[
        pltpu.VMEM((CMAX * CH,), jnp.int32),
        pltpu.VMEM((CMAX * CH,), jnp.int32),
        pltpu.VMEM((CH, HID), jnp.float32),
        pltpu.VMEM((CH, HID), jnp.float32),
        pltpu.SemaphoreType.DMA,
        pltpu.SemaphoreType.DMA,
        pltpu.VMEM_SHARED((SROWS, HID), jnp.float32),
        pltpu.VMEM_SHARED((SROWS, HID), jnp.float32),
    ],
    compiler_params=_sc_params,
)
def _msg_kernel(ei_hbm, y_hbm, zeros_hbm, s_out,
                src_v, dst_v, buf0, buf1, sem0, sem1, acc_sh, y_sh):
    cid = lax.axis_index("c")
    sid = lax.axis_index("s")
    w, row_start, ncap = _worker_bounds(cid, sid)
    src_hbm = ei_hbm.at[0]
    pltpu.sync_copy(src_hbm.at[pl.ds(row_start * CH, BASE * CH)],
                    src_v.at[pl.ds(0, BASE * CH)])

    @pl.when(w < EXTRA)
    def _():
        pltpu.sync_copy(src_hbm.at[pl.ds((row_start + BASE) * CH, CH)],
                        src_v.at[pl.ds(BASE * CH, CH)])

    _dst_slab(ei_hbm.at[1], dst_v, row_start, w)

    pltpu.sync_copy(y_hbm.at[pl.ds(sid * RPT, RPT)],
                    y_sh.at[pl.ds(sid * RPT, RPT)])
    pltpu.sync_copy(zeros_hbm, acc_sh.at[pl.ds(sid * RPT, RPT)])
    plsc.subcore_barrier()

    def src_idx(j):
        return src_v.at[pl.ds(j * CH, CH)]

    pltpu.async_copy(y_sh.at[src_idx(0)], buf0, sem0)
    pltpu.async_copy(y_sh.at[src_idx(1)], buf1, sem1)

    def pair(i, _):
        j0 = i * 2
        for b, (buf, sem) in enumerate(((buf0, sem0), (buf1, sem1))):
            j = j0 + b
            pltpu.make_async_copy(y_sh.at[src_idx(j)], buf, sem).wait()
            pltpu.sync_copy(buf, acc_sh.at[dst_v.at[pl.ds(j * CH, CH)]], add=True)

            @pl.when(j + 2 < ncap)
            def _():
                pltpu.async_copy(y_sh.at[src_idx(j + 2)], buf, sem)

        return ()

    lax.fori_loop(0, BASE // 2, pair, ())

    @pl.when(ncap > BASE)
    def _():
        pltpu.make_async_copy(y_sh.at[src_idx(BASE)], buf0, sem0).wait()
        pltpu.sync_copy(buf0, acc_sh.at[dst_v.at[pl.ds(BASE * CH, CH)]], add=True)

    plsc.subcore_barrier()
    pltpu.sync_copy(acc_sh.at[pl.ds(sid * RPT, RPT)],
                    s_out.at[cid, pl.ds(sid * RPT, RPT)])


def _proj_body(x_ref, wg_ref, wps_ref, bps_ref, xw_ref, zsem_ref):
    x = x_ref[...]
    xw_ref[...] = jnp.dot(x, wg_ref[...], preferred_element_type=jnp.float32)
    zsem_ref[...] = (
        jnp.dot(x, wps_ref[...], preferred_element_type=jnp.float32)
        + bps_ref[...]
    )


def _proj(x, W_gcn, W_ps, b_ps):
    return pl.pallas_call(
        _proj_body,
        grid=(GRID,),
        in_specs=[
            pl.BlockSpec((BR, IN_DIM), lambda i: (i, 0)),
            pl.BlockSpec((IN_DIM, HID), lambda i: (0, 0)),
            pl.BlockSpec((IN_DIM, ALIGN), lambda i: (0, 0)),
            pl.BlockSpec((1, ALIGN), lambda i: (0, 0)),
        ],
        out_specs=[
            pl.BlockSpec((BR, HID), lambda i: (i, 0)),
            pl.BlockSpec((BR, ALIGN), lambda i: (i, 0)),
        ],
        out_shape=[
            jax.ShapeDtypeStruct((N, HID), jnp.float32),
            jax.ShapeDtypeStruct((N, ALIGN), jnp.float32),
        ],
    )(x, W_gcn, W_ps, b_ps)


def _scale_body(dp_ref, xw_ref, y_ref, dinv_ref):
    deg = dp_ref[0, :, 0:1] + dp_ref[1, :, 0:1] + 1.0
    dinv = lax.rsqrt(deg)
    y_ref[...] = dinv * xw_ref[...]
    dinv_ref[...] = jnp.broadcast_to(dinv, dinv_ref.shape)


def _scale(deg_parts, xw):
    return pl.pallas_call(
        _scale_body,
        grid=(GRID,),
        in_specs=[
            pl.BlockSpec((2, BR, 16), lambda i: (0, i, 0)),
            pl.BlockSpec((BR, HID), lambda i: (i, 0)),
        ],
        out_specs=[
            pl.BlockSpec((BR, HID), lambda i: (i, 0)),
            pl.BlockSpec((BR, 16), lambda i: (i, 0)),
        ],
        out_shape=[
            jax.ShapeDtypeStruct((SROWS, HID), jnp.float32),
            jax.ShapeDtypeStruct((N, 16), jnp.float32),
        ],
    )(deg_parts, xw)


def _head_body(s_ref, y_ref, dinv_ref, zsem_ref, bg_ref, wpt_ref, bpt_ref,
               wcls_ref, bcls_ref, logits_ref, an_ref, ztopo_ref):
    dinv = dinv_ref[:, 0:1]
    s_tot = s_ref[0] + s_ref[1] + y_ref[...]
    h = jnp.maximum(dinv * s_tot + bg_ref[...], 0.0)
    z_topo = (
        jnp.dot(h, wpt_ref[...], preferred_element_type=jnp.float32)
        + bpt_ref[...]
    )
    logits_ref[...] = (
        jnp.dot(z_topo, wcls_ref[...], preferred_element_type=jnp.float32)
        + bcls_ref[...]
    )
    diff = z_topo - zsem_ref[...]
    an_ref[...] = jnp.sqrt(jnp.sum(diff * diff, axis=1))
    ztopo_ref[...] = z_topo


def _heads(s_parts, y, dinv, z_sem, b_gcn, W_pt, b_pt, W_cls, b_cls):
    return pl.pallas_call(
        _head_body,
        grid=(GRID,),
        in_specs=[
            pl.BlockSpec((2, BR, HID), lambda i: (0, i, 0)),
            pl.BlockSpec((BR, HID), lambda i: (i, 0)),
            pl.BlockSpec((BR, 16), lambda i: (i, 0)),
            pl.BlockSpec((BR, ALIGN), lambda i: (i, 0)),
            pl.BlockSpec((1, HID), lambda i: (0, 0)),
            pl.BlockSpec((HID, ALIGN), lambda i: (0, 0)),
            pl.BlockSpec((1, ALIGN), lambda i: (0, 0)),
            pl.BlockSpec((ALIGN, NUM_CLASSES), lambda i: (0, 0)),
            pl.BlockSpec((1, NUM_CLASSES), lambda i: (0, 0)),
        ],
        out_specs=[
            pl.BlockSpec((BR, NUM_CLASSES), lambda i: (i, 0)),
            pl.BlockSpec((BR,), lambda i: (i,)),
            pl.BlockSpec((BR, ALIGN), lambda i: (i, 0)),
        ],
        out_shape=[
            jax.ShapeDtypeStruct((N, NUM_CLASSES), jnp.float32),
            jax.ShapeDtypeStruct((N,), jnp.float32),
            jax.ShapeDtypeStruct((N, ALIGN), jnp.float32),
        ],
    )(s_parts, y, dinv, z_sem, b_gcn, W_pt, b_pt, W_cls, b_cls)


def kernel(x, edge_index, W_gcn, b_gcn, W_pt, b_pt, W_ps, b_ps, W_cls, b_cls):

    zeros16 = jnp.zeros((RPT, 16), jnp.float32)
    zeros64 = jnp.zeros((RPT, HID), jnp.float32)
    ones16 = jnp.ones((CH, 16), jnp.float32)

    deg_parts = _deg_kernel(edge_index, zeros16, ones16)
    xw, z_sem = _proj(x, W_gcn, W_ps, b_ps.reshape(1, ALIGN))
    y, dinv = _scale(deg_parts, xw)
    s_parts = _msg_kernel(edge_index, y, zeros64)
    logits, anomaly, z_topo = _heads(
        s_parts, y, dinv, z_sem, b_gcn.reshape(1, HID), W_pt,
        b_pt.reshape(1, ALIGN), W_cls, b_cls.reshape(1, NUM_CLASSES))
    return (logits, anomaly, z_topo, z_sem)


# speedup vs baseline: 1.0626x; 1.0043x over previous
import functools

import jax
import jax.numpy as jnp
from jax import lax
from jax.experimental import pallas as pl
from jax.experimental.pallas import tpu as pltpu
from jax.experimental.pallas import tpu_sc as plsc

N = 10000
E = 320000
IN_DIM = 128
HID = 64
ALIGN = 32
NUM_CLASSES = 7

NC = 2
NS = 16
NW = NC * NS
CH = 128
NCHUNK = E // CH
BASE = NCHUNK // NW
EXTRA = NCHUNK - BASE * NW
CMAX = BASE + 1
SROWS = 10240
RPT = SROWS // NS

BR = 2048
GRID = (N + BR - 1) // BR

_mesh = plsc.VectorSubcoreMesh(core_axis_name="c", subcore_axis_name="s")
_sc_params = pltpu.CompilerParams(use_tc_tiling_on_sc=False)


def _worker_bounds(cid, sid):
    w = cid * NS + sid
    row_start = w * BASE + jnp.minimum(w, EXTRA)
    ncap = BASE + jnp.where(w < EXTRA, 1, 0)
    return w, row_start, ncap


def _dst_slab(dst_hbm, dst_v, row_start, w):
    pltpu.sync_copy(dst_hbm.at[pl.ds(row_start * CH, BASE * CH)],
                    dst_v.at[pl.ds(0, BASE * CH)])

    @pl.when(w < EXTRA)
    def _():
        pltpu.sync_copy(dst_hbm.at[pl.ds((row_start + BASE) * CH, CH)],
                        dst_v.at[pl.ds(BASE * CH, CH)])


@functools.partial(
    pl.kernel,
    out_type=jax.ShapeDtypeStruct((NC, SROWS, 16), jnp.float32),
    mesh=_mesh,
    scratch_types=[
        pltpu.VMEM((CMAX * CH,), jnp.int32),
        pltpu.VMEM((CH, 16), jnp.float32),
        pltpu.VMEM_SHARED((SROWS, 16), jnp.float32),
    ],
    compiler_params=_sc_params,
)
def _deg_kernel(ei_hbm, zeros_hbm, ones_hbm, deg_out, dst_v, ones_v, acc_sh):
    cid = lax.axis_index("c")
    sid = lax.axis_index("s")
    w, row_start, ncap = _worker_bounds(cid, sid)
    _dst_slab(ei_hbm.at[1], dst_v, row_start, w)
    pltpu.sync_copy(ones_hbm, ones_v)

    pltpu.sync_copy(zeros_hbm, acc_sh.at[pl.ds(sid * RPT, RPT)])
    plsc.subcore_barrier()

    def chunk(j, _):
        pltpu.sync_copy(ones_v, acc_sh.at[dst_v.at[pl.ds(j * CH, CH)]], add=True)
        return ()

    lax.fori_loop(0, ncap, chunk, ())
    plsc.subcore_barrier()
    pltpu.sync_copy(acc_sh.at[pl.ds(sid * RPT, RPT)],
                    deg_out.at[cid, pl.ds(sid * RPT, RPT)])


@functools.partial(
    pl.kernel,
    out_type=jax.ShapeDtypeStruct((NC, SROWS, HID), jnp.float32),
    mesh=_mesh,
    scratch_types=[
        pltpu.VMEM((CMAX * CH,), jnp.int32),
        pltpu.VMEM((CMAX * CH,), jnp.int32),
        pltpu.VMEM((CH, HID), jnp.float32),
        pltpu.VMEM((CH, HID), jnp.float32),
        pltpu.VMEM((CH, HID), jnp.float32),
        pltpu.SemaphoreType.DMA,
        pltpu.SemaphoreType.DMA,
        pltpu.SemaphoreType.DMA,
        pltpu.SemaphoreType.DMA,
        pltpu.SemaphoreType.DMA,
        pltpu.SemaphoreType.DMA,
        pltpu.VMEM_SHARED((SROWS, HID), jnp.float32),
        pltpu.VMEM_SHARED((SROWS, HID), jnp.float32),
    ],
    compiler_params=_sc_params,
)
def _msg_kernel(ei_hbm, y_hbm, zeros_hbm, s_out,
                src_v, dst_v, buf0, buf1, buf2,
                gsem0, gsem1, gsem2, ssem0, ssem1, ssem2, acc_sh, y_sh):
    cid = lax.axis_index("c")
    sid = lax.axis_index("s")
    w, row_start, ncap = _worker_bounds(cid, sid)
    src_hbm = ei_hbm.at[0]
    pltpu.sync_copy(src_hbm.at[pl.ds(row_start * CH, BASE * CH)],
                    src_v.at[pl.ds(0, BASE * CH)])

    @pl.when(w < EXTRA)
    def _():
        pltpu.sync_copy(src_hbm.at[pl.ds((row_start + BASE) * CH, CH)],
                        src_v.at[pl.ds(BASE * CH, CH)])

    _dst_slab(ei_hbm.at[1], dst_v, row_start, w)

    pltpu.sync_copy(y_hbm.at[pl.ds(sid * RPT, RPT)],
                    y_sh.at[pl.ds(sid * RPT, RPT)])
    pltpu.sync_copy(zeros_hbm, acc_sh.at[pl.ds(sid * RPT, RPT)])
    plsc.subcore_barrier()

    def src_idx(j):
        return src_v.at[pl.ds(j * CH, CH)]

    def dst_idx(j):
        return dst_v.at[pl.ds(j * CH, CH)]

    bufs = (buf0, buf1, buf2)
    gsems = (gsem0, gsem1, gsem2)
    ssems = (ssem0, ssem1, ssem2)

    pltpu.async_copy(y_sh.at[src_idx(0)], buf0, gsem0)
    pltpu.async_copy(y_sh.at[src_idx(1)], buf1, gsem1)

    def trip(i, _):
        j0 = i * 3
        for b in range(3):
            j = j0 + b
            buf, gsem, ssem = bufs[b], gsems[b], ssems[b]
            b2 = (b + 2) % 3
            pltpu.make_async_copy(y_sh.at[src_idx(j)], buf, gsem).wait()
            if b == 0:
                @pl.when(i >= 1)
                def _():
                    pltpu.make_async_copy(
                        bufs[b2], acc_sh.at[dst_idx(j)], ssems[b2]).wait()
            else:
                pltpu.make_async_copy(
                    bufs[b2], acc_sh.at[dst_idx(j)], ssems[b2]).wait()
            pltpu.async_copy(buf, acc_sh.at[dst_idx(j)], ssem, add=True)

            @pl.when(j + 2 < ncap)
            def _():
                pltpu.async_copy(y_sh.at[src_idx(j + 2)], bufs[b2], gsems[b2])

        return ()

    lax.fori_loop(0, BASE // 3, trip, ())

    pltpu.make_async_copy(buf2, acc_sh.at[dst_idx(BASE - 1)], ssem2).wait()

    @pl.when(ncap > BASE)
    def _():
        pltpu.make_async_copy(y_sh.at[src_idx(BASE)], bufs[BASE % 3],
                              gsems[BASE % 3]).wait()
        pltpu.sync_copy(bufs[BASE % 3], acc_sh.at[dst_idx(BASE)], add=True)

    plsc.subcore_barrier()
    pltpu.sync_copy(acc_sh.at[pl.ds(sid * RPT, RPT)],
                    s_out.at[cid, pl.ds(sid * RPT, RPT)])


def _proj_body(x_ref, wg_ref, wps_ref, bps_ref, xw_ref, zsem_ref):
    x = x_ref[...]
    xw_ref[...] = jnp.dot(x, wg_ref[...], preferred_element_type=jnp.float32)
    zsem_ref[...] = (
        jnp.dot(x, wps_ref[...], preferred_element_type=jnp.float32)
        + bps_ref[...]
    )


def _proj(x, W_gcn, W_ps, b_ps):
    return pl.pallas_call(
        _proj_body,
        grid=(GRID,),
        in_specs=[
            pl.BlockSpec((BR, IN_DIM), lambda i: (i, 0)),
            pl.BlockSpec((IN_DIM, HID), lambda i: (0, 0)),
            pl.BlockSpec((IN_DIM, ALIGN), lambda i: (0, 0)),
            pl.BlockSpec((1, ALIGN), lambda i: (0, 0)),
        ],
        out_specs=[
            pl.BlockSpec((BR, HID), lambda i: (i, 0)),
            pl.BlockSpec((BR, ALIGN), lambda i: (i, 0)),
        ],
        out_shape=[
            jax.ShapeDtypeStruct((N, HID), jnp.float32),
            jax.ShapeDtypeStruct((N, ALIGN), jnp.float32),
        ],
    )(x, W_gcn, W_ps, b_ps)


def _scale_body(dp_ref, xw_ref, y_ref, dinv_ref):
    deg = dp_ref[0, :, 0:1] + dp_ref[1, :, 0:1] + 1.0
    dinv = lax.rsqrt(deg)
    y_ref[...] = dinv * xw_ref[...]
    dinv_ref[...] = jnp.broadcast_to(dinv, dinv_ref.shape)


def _scale(deg_parts, xw):
    return pl.pallas_call(
        _scale_body,
        grid=(GRID,),
        in_specs=[
            pl.BlockSpec((2, BR, 16), lambda i: (0, i, 0)),
            pl.BlockSpec((BR, HID), lambda i: (i, 0)),
        ],
        out_specs=[
            pl.BlockSpec((BR, HID), lambda i: (i, 0)),
            pl.BlockSpec((BR, 16), lambda i: (i, 0)),
        ],
        out_shape=[
            jax.ShapeDtypeStruct((SROWS, HID), jnp.float32),
            jax.ShapeDtypeStruct((N, 16), jnp.float32),
        ],
    )(deg_parts, xw)


def _head_body(s_ref, y_ref, dinv_ref, zsem_ref, bg_ref, wpt_ref, bpt_ref,
               wcls_ref, bcls_ref, logits_ref, an_ref, ztopo_ref):
    dinv = dinv_ref[:, 0:1]
    s_tot = s_ref[0] + s_ref[1] + y_ref[...]
    h = jnp.maximum(dinv * s_tot + bg_ref[...], 0.0)
    z_topo = (
        jnp.dot(h, wpt_ref[...], preferred_element_type=jnp.float32)
        + bpt_ref[...]
    )
    logits_ref[...] = (
        jnp.dot(z_topo, wcls_ref[...], preferred_element_type=jnp.float32)
        + bcls_ref[...]
    )
    diff = z_topo - zsem_ref[...]
    an_ref[...] = jnp.sqrt(jnp.sum(diff * diff, axis=1))
    ztopo_ref[...] = z_topo


def _heads(s_parts, y, dinv, z_sem, b_gcn, W_pt, b_pt, W_cls, b_cls):
    return pl.pallas_call(
        _head_body,
        grid=(GRID,),
        in_specs=[
            pl.BlockSpec((2, BR, HID), lambda i: (0, i, 0)),
            pl.BlockSpec((BR, HID), lambda i: (i, 0)),
            pl.BlockSpec((BR, 16), lambda i: (i, 0)),
            pl.BlockSpec((BR, ALIGN), lambda i: (i, 0)),
            pl.BlockSpec((1, HID), lambda i: (0, 0)),
            pl.BlockSpec((HID, ALIGN), lambda i: (0, 0)),
            pl.BlockSpec((1, ALIGN), lambda i: (0, 0)),
            pl.BlockSpec((ALIGN, NUM_CLASSES), lambda i: (0, 0)),
            pl.BlockSpec((1, NUM_CLASSES), lambda i: (0, 0)),
        ],
        out_specs=[
            pl.BlockSpec((BR, NUM_CLASSES), lambda i: (i, 0)),
            pl.BlockSpec((BR,), lambda i: (i,)),
            pl.BlockSpec((BR, ALIGN), lambda i: (i, 0)),
        ],
        out_shape=[
            jax.ShapeDtypeStruct((N, NUM_CLASSES), jnp.float32),
            jax.ShapeDtypeStruct((N,), jnp.float32),
            jax.ShapeDtypeStruct((N, ALIGN), jnp.float32),
        ],
    )(s_parts, y, dinv, z_sem, b_gcn, W_pt, b_pt, W_cls, b_cls)


def kernel(x, edge_index, W_gcn, b_gcn, W_pt, b_pt, W_ps, b_ps, W_cls, b_cls):

    zeros16 = jnp.zeros((RPT, 16), jnp.float32)
    zeros64 = jnp.zeros((RPT, HID), jnp.float32)
    ones16 = jnp.ones((CH, 16), jnp.float32)

    deg_parts = _deg_kernel(edge_index, zeros16, ones16)
    xw, z_sem = _proj(x, W_gcn, W_ps, b_ps.reshape(1, ALIGN))
    y, dinv = _scale(deg_parts, xw)
    s_parts = _msg_kernel(edge_index, y, zeros64)
    logits, anomaly, z_topo = _heads(
        s_parts, y, dinv, z_sem, b_gcn.reshape(1, HID), W_pt,
        b_pt.reshape(1, ALIGN), W_cls, b_cls.reshape(1, NUM_CLASSES))
    return (logits, anomaly, z_topo, z_sem)


# speedup vs baseline: 1.0823x; 1.0185x over previous
import functools

import jax
import jax.numpy as jnp
from jax import lax
from jax.experimental import pallas as pl
from jax.experimental.pallas import tpu as pltpu
from jax.experimental.pallas import tpu_sc as plsc

N = 10000
E = 320000
IN_DIM = 128
HID = 64
ALIGN = 32
NUM_CLASSES = 7

NC = 2
NS = 16
NW = NC * NS
CH = 128
NCHUNK = E // CH
BASE = NCHUNK // NW
EXTRA = NCHUNK - BASE * NW
CMAX = BASE + 1
SROWS = 10240
RPT = SROWS // NS

BR = 2048
GRID = (N + BR - 1) // BR

_mesh = plsc.VectorSubcoreMesh(core_axis_name="c", subcore_axis_name="s")
_sc_params = pltpu.CompilerParams(use_tc_tiling_on_sc=False)


def _worker_bounds(cid, sid):
    w = cid * NS + sid
    row_start = w * BASE + jnp.minimum(w, EXTRA)
    ncap = BASE + jnp.where(w < EXTRA, 1, 0)
    return w, row_start, ncap


def _dst_slab(dst_hbm, dst_v, row_start, w):
    pltpu.sync_copy(dst_hbm.at[pl.ds(row_start * CH, BASE * CH)],
                    dst_v.at[pl.ds(0, BASE * CH)])

    @pl.when(w < EXTRA)
    def _():
        pltpu.sync_copy(dst_hbm.at[pl.ds((row_start + BASE) * CH, CH)],
                        dst_v.at[pl.ds(BASE * CH, CH)])


@functools.partial(
    pl.kernel,
    out_type=jax.ShapeDtypeStruct((NC, SROWS, 8), jnp.float32),
    mesh=_mesh,
    scratch_types=[
        pltpu.VMEM((CMAX * CH,), jnp.int32),
        pltpu.VMEM((CH, 8), jnp.float32),
        pltpu.VMEM_SHARED((SROWS, 8), jnp.float32),
    ],
    compiler_params=_sc_params,
)
def _deg_kernel(ei_hbm, zeros_hbm, ones_hbm, deg_out, dst_v, ones_v, acc_sh):
    cid = lax.axis_index("c")
    sid = lax.axis_index("s")
    w, row_start, ncap = _worker_bounds(cid, sid)
    _dst_slab(ei_hbm.at[1], dst_v, row_start, w)
    pltpu.sync_copy(ones_hbm, ones_v)

    pltpu.sync_copy(zeros_hbm, acc_sh.at[pl.ds(sid * RPT, RPT)])
    plsc.subcore_barrier()

    def chunk(j, _):
        pltpu.sync_copy(ones_v, acc_sh.at[dst_v.at[pl.ds(j * CH, CH)]], add=True)
        return ()

    lax.fori_loop(0, ncap, chunk, ())
    plsc.subcore_barrier()
    pltpu.sync_copy(acc_sh.at[pl.ds(sid * RPT, RPT)],
                    deg_out.at[cid, pl.ds(sid * RPT, RPT)])


@functools.partial(
    pl.kernel,
    out_type=jax.ShapeDtypeStruct((NC, SROWS, HID), jnp.float32),
    mesh=_mesh,
    scratch_types=[
        pltpu.VMEM((CMAX * CH,), jnp.int32),
        pltpu.VMEM((CMAX * CH,), jnp.int32),
        pltpu.VMEM((CH, HID), jnp.float32),
        pltpu.VMEM((CH, HID), jnp.float32),
        pltpu.VMEM((CH, HID), jnp.float32),
        pltpu.SemaphoreType.DMA,
        pltpu.SemaphoreType.DMA,
        pltpu.SemaphoreType.DMA,
        pltpu.SemaphoreType.DMA,
        pltpu.SemaphoreType.DMA,
        pltpu.SemaphoreType.DMA,
        pltpu.VMEM_SHARED((SROWS, HID), jnp.float32),
        pltpu.VMEM_SHARED((SROWS, HID), jnp.float32),
    ],
    compiler_params=_sc_params,
)
def _msg_kernel(ei_hbm, y_hbm, zeros_hbm, s_out,
                src_v, dst_v, buf0, buf1, buf2,
                gsem0, gsem1, gsem2, ssem0, ssem1, ssem2, acc_sh, y_sh):
    cid = lax.axis_index("c")
    sid = lax.axis_index("s")
    w, row_start, ncap = _worker_bounds(cid, sid)
    src_hbm = ei_hbm.at[0]
    pltpu.sync_copy(src_hbm.at[pl.ds(row_start * CH, BASE * CH)],
                    src_v.at[pl.ds(0, BASE * CH)])

    @pl.when(w < EXTRA)
    def _():
        pltpu.sync_copy(src_hbm.at[pl.ds((row_start + BASE) * CH, CH)],
                        src_v.at[pl.ds(BASE * CH, CH)])

    _dst_slab(ei_hbm.at[1], dst_v, row_start, w)

    pltpu.sync_copy(y_hbm.at[pl.ds(sid * RPT, RPT)],
                    y_sh.at[pl.ds(sid * RPT, RPT)])
    pltpu.sync_copy(zeros_hbm, acc_sh.at[pl.ds(sid * RPT, RPT)])
    plsc.subcore_barrier()

    def src_idx(j):
        return src_v.at[pl.ds(j * CH, CH)]

    def dst_idx(j):
        return dst_v.at[pl.ds(j * CH, CH)]

    bufs = (buf0, buf1, buf2)
    gsems = (gsem0, gsem1, gsem2)
    ssems = (ssem0, ssem1, ssem2)

    pltpu.async_copy(y_sh.at[src_idx(0)], buf0, gsem0)
    pltpu.async_copy(y_sh.at[src_idx(1)], buf1, gsem1)

    def trip(i, _):
        j0 = i * 3
        for b in range(3):
            j = j0 + b
            buf, gsem, ssem = bufs[b], gsems[b], ssems[b]
            b2 = (b + 2) % 3
            pltpu.make_async_copy(y_sh.at[src_idx(j)], buf, gsem).wait()
            if b == 0:
                @pl.when(i >= 1)
                def _():
                    pltpu.make_async_copy(
                        bufs[b2], acc_sh.at[dst_idx(j)], ssems[b2]).wait()
            else:
                pltpu.make_async_copy(
                    bufs[b2], acc_sh.at[dst_idx(j)], ssems[b2]).wait()
            pltpu.async_copy(buf, acc_sh.at[dst_idx(j)], ssem, add=True)

            @pl.when(j + 2 < ncap)
            def _():
                pltpu.async_copy(y_sh.at[src_idx(j + 2)], bufs[b2], gsems[b2])

        return ()

    lax.fori_loop(0, BASE // 3, trip, ())

    pltpu.make_async_copy(buf2, acc_sh.at[dst_idx(BASE - 1)], ssem2).wait()

    @pl.when(ncap > BASE)
    def _():
        pltpu.make_async_copy(y_sh.at[src_idx(BASE)], bufs[BASE % 3],
                              gsems[BASE % 3]).wait()
        pltpu.sync_copy(bufs[BASE % 3], acc_sh.at[dst_idx(BASE)], add=True)

    plsc.subcore_barrier()
    pltpu.sync_copy(acc_sh.at[pl.ds(sid * RPT, RPT)],
                    s_out.at[cid, pl.ds(sid * RPT, RPT)])


def _proj_body(x_ref, wg_ref, wps_ref, bps_ref, xw_ref, zsem_ref):
    x = x_ref[...]
    xw_ref[...] = jnp.dot(x, wg_ref[...], preferred_element_type=jnp.float32)
    zsem_ref[...] = (
        jnp.dot(x, wps_ref[...], preferred_element_type=jnp.float32)
        + bps_ref[...]
    )


def _proj(x, W_gcn, W_ps, b_ps):
    return pl.pallas_call(
        _proj_body,
        grid=(GRID,),
        in_specs=[
            pl.BlockSpec((BR, IN_DIM), lambda i: (i, 0)),
            pl.BlockSpec((IN_DIM, HID), lambda i: (0, 0)),
            pl.BlockSpec((IN_DIM, ALIGN), lambda i: (0, 0)),
            pl.BlockSpec((1, ALIGN), lambda i: (0, 0)),
        ],
        out_specs=[
            pl.BlockSpec((BR, HID), lambda i: (i, 0)),
            pl.BlockSpec((BR, ALIGN), lambda i: (i, 0)),
        ],
        out_shape=[
            jax.ShapeDtypeStruct((N, HID), jnp.float32),
            jax.ShapeDtypeStruct((N, ALIGN), jnp.float32),
        ],
    )(x, W_gcn, W_ps, b_ps)


def _scale_body(dp_ref, xw_ref, y_ref, dinv_ref):
    deg = dp_ref[0, :, 0:1] + dp_ref[1, :, 0:1] + 1.0
    dinv = lax.rsqrt(deg)
    y_ref[...] = dinv * xw_ref[...]
    dinv_ref[...] = jnp.broadcast_to(dinv, dinv_ref.shape)


def _scale(deg_parts, xw):
    return pl.pallas_call(
        _scale_body,
        grid=(GRID,),
        in_specs=[
            pl.BlockSpec((2, BR, 8), lambda i: (0, i, 0)),
            pl.BlockSpec((BR, HID), lambda i: (i, 0)),
        ],
        out_specs=[
            pl.BlockSpec((BR, HID), lambda i: (i, 0)),
            pl.BlockSpec((BR, 16), lambda i: (i, 0)),
        ],
        out_shape=[
            jax.ShapeDtypeStruct((SROWS, HID), jnp.float32),
            jax.ShapeDtypeStruct((N, 16), jnp.float32),
        ],
    )(deg_parts, xw)


def _head_body(s_ref, y_ref, dinv_ref, zsem_ref, bg_ref, wpt_ref, bpt_ref,
               wcls_ref, bcls_ref, logits_ref, an_ref, ztopo_ref):
    dinv = dinv_ref[:, 0:1]
    s_tot = s_ref[0] + s_ref[1] + y_ref[...]
    h = jnp.maximum(dinv * s_tot + bg_ref[...], 0.0)
    z_topo = (
        jnp.dot(h, wpt_ref[...], preferred_element_type=jnp.float32)
        + bpt_ref[...]
    )
    logits_ref[...] = (
        jnp.dot(z_topo, wcls_ref[...], preferred_element_type=jnp.float32)
        + bcls_ref[...]
    )
    diff = z_topo - zsem_ref[...]
    an_ref[...] = jnp.sqrt(jnp.sum(diff * diff, axis=1))
    ztopo_ref[...] = z_topo


def _heads(s_parts, y, dinv, z_sem, b_gcn, W_pt, b_pt, W_cls, b_cls):
    return pl.pallas_call(
        _head_body,
        grid=(GRID,),
        in_specs=[
            pl.BlockSpec((2, BR, HID), lambda i: (0, i, 0)),
            pl.BlockSpec((BR, HID), lambda i: (i, 0)),
            pl.BlockSpec((BR, 16), lambda i: (i, 0)),
            pl.BlockSpec((BR, ALIGN), lambda i: (i, 0)),
            pl.BlockSpec((1, HID), lambda i: (0, 0)),
            pl.BlockSpec((HID, ALIGN), lambda i: (0, 0)),
            pl.BlockSpec((1, ALIGN), lambda i: (0, 0)),
            pl.BlockSpec((ALIGN, NUM_CLASSES), lambda i: (0, 0)),
            pl.BlockSpec((1, NUM_CLASSES), lambda i: (0, 0)),
        ],
        out_specs=[
            pl.BlockSpec((BR, NUM_CLASSES), lambda i: (i, 0)),
            pl.BlockSpec((BR,), lambda i: (i,)),
            pl.BlockSpec((BR, ALIGN), lambda i: (i, 0)),
        ],
        out_shape=[
            jax.ShapeDtypeStruct((N, NUM_CLASSES), jnp.float32),
            jax.ShapeDtypeStruct((N,), jnp.float32),
            jax.ShapeDtypeStruct((N, ALIGN), jnp.float32),
        ],
    )(s_parts, y, dinv, z_sem, b_gcn, W_pt, b_pt, W_cls, b_cls)


def kernel(x, edge_index, W_gcn, b_gcn, W_pt, b_pt, W_ps, b_ps, W_cls, b_cls):

    zeros16 = jnp.zeros((RPT, 8), jnp.float32)
    zeros64 = jnp.zeros((RPT, HID), jnp.float32)
    ones16 = jnp.ones((CH, 8), jnp.float32)

    deg_parts = _deg_kernel(edge_index, zeros16, ones16)
    xw, z_sem = _proj(x, W_gcn, W_ps, b_ps.reshape(1, ALIGN))
    y, dinv = _scale(deg_parts, xw)
    s_parts = _msg_kernel(edge_index, y, zeros64)
    logits, anomaly, z_topo = _heads(
        s_parts, y, dinv, z_sem, b_gcn.reshape(1, HID), W_pt,
        b_pt.reshape(1, ALIGN), W_cls, b_cls.reshape(1, NUM_CLASSES))
    return (logits, anomaly, z_topo, z_sem)


# speedup vs baseline: 1.0950x; 1.0118x over previous
import functools

import jax
import jax.numpy as jnp
from jax import lax
from jax.experimental import pallas as pl
from jax.experimental.pallas import tpu as pltpu
from jax.experimental.pallas import tpu_sc as plsc

N = 10000
E = 320000
IN_DIM = 128
HID = 64
ALIGN = 32
NUM_CLASSES = 7

NC = 2
NS = 16
NW = NC * NS
CH = 128
NCHUNK = E // CH
BASE = NCHUNK // NW
EXTRA = NCHUNK - BASE * NW
CMAX = BASE + 1
SROWS = 10240
RPT = SROWS // NS

BR = 2048
GRID = (N + BR - 1) // BR

_mesh = plsc.VectorSubcoreMesh(core_axis_name="c", subcore_axis_name="s")
_sc_params = pltpu.CompilerParams(use_tc_tiling_on_sc=False)


def _worker_bounds(cid, sid):
    w = cid * NS + sid
    row_start = w * BASE + jnp.minimum(w, EXTRA)
    ncap = BASE + jnp.where(w < EXTRA, 1, 0)
    return w, row_start, ncap


def _dst_slab(dst_hbm, dst_v, row_start, w):
    pltpu.sync_copy(dst_hbm.at[pl.ds(row_start * CH, BASE * CH)],
                    dst_v.at[pl.ds(0, BASE * CH)])

    @pl.when(w < EXTRA)
    def _():
        pltpu.sync_copy(dst_hbm.at[pl.ds((row_start + BASE) * CH, CH)],
                        dst_v.at[pl.ds(BASE * CH, CH)])


@functools.partial(
    pl.kernel,
    out_type=jax.ShapeDtypeStruct((NC, SROWS, 8), jnp.float32),
    mesh=_mesh,
    scratch_types=[
        pltpu.VMEM((CMAX * CH,), jnp.int32),
        pltpu.VMEM((CH, 8), jnp.float32),
        pltpu.SemaphoreType.DMA,
        pltpu.VMEM_SHARED((SROWS, 8), jnp.float32),
    ],
    compiler_params=_sc_params,
)
def _deg_kernel(ei_hbm, zeros_hbm, ones_hbm, deg_out, dst_v, ones_v, sem,
                acc_sh):
    cid = lax.axis_index("c")
    sid = lax.axis_index("s")
    w, row_start, ncap = _worker_bounds(cid, sid)
    _dst_slab(ei_hbm.at[1], dst_v, row_start, w)
    pltpu.sync_copy(ones_hbm, ones_v)

    pltpu.sync_copy(zeros_hbm, acc_sh.at[pl.ds(sid * RPT, RPT)])
    plsc.subcore_barrier()

    def chunk(j, _):
        pltpu.async_copy(ones_v, acc_sh.at[dst_v.at[pl.ds(j * CH, CH)]],
                         sem, add=True)

        @pl.when(j >= 4)
        def _():
            pltpu.make_async_copy(
                ones_v, acc_sh.at[dst_v.at[pl.ds(0, CH)]], sem).wait()

        return ()

    lax.fori_loop(0, ncap, chunk, ())
    for _ in range(4):
        pltpu.make_async_copy(
            ones_v, acc_sh.at[dst_v.at[pl.ds(0, CH)]], sem).wait()
    plsc.subcore_barrier()
    pltpu.sync_copy(acc_sh.at[pl.ds(sid * RPT, RPT)],
                    deg_out.at[cid, pl.ds(sid * RPT, RPT)])


@functools.partial(
    pl.kernel,
    out_type=jax.ShapeDtypeStruct((NC, SROWS, HID), jnp.float32),
    mesh=_mesh,
    scratch_types=[
        pltpu.VMEM((CMAX * CH,), jnp.int32),
        pltpu.VMEM((CMAX * CH,), jnp.int32),
        pltpu.VMEM((CH, HID), jnp.float32),
        pltpu.VMEM((CH, HID), jnp.float32),
        pltpu.VMEM((CH, HID), jnp.float32),
        pltpu.SemaphoreType.DMA,
        pltpu.SemaphoreType.DMA,
        pltpu.SemaphoreType.DMA,
        pltpu.SemaphoreType.DMA,
        pltpu.SemaphoreType.DMA,
        pltpu.SemaphoreType.DMA,
        pltpu.VMEM_SHARED((SROWS, HID), jnp.float32),
        pltpu.VMEM_SHARED((SROWS, HID), jnp.float32),
    ],
    compiler_params=_sc_params,
)
def _msg_kernel(ei_hbm, y_hbm, zeros_hbm, s_out,
                src_v, dst_v, buf0, buf1, buf2,
                gsem0, gsem1, gsem2, ssem0, ssem1, ssem2, acc_sh, y_sh):
    cid = lax.axis_index("c")
    sid = lax.axis_index("s")
    w, row_start, ncap = _worker_bounds(cid, sid)
    src_hbm = ei_hbm.at[0]
    pltpu.sync_copy(src_hbm.at[pl.ds(row_start * CH, BASE * CH)],
                    src_v.at[pl.ds(0, BASE * CH)])

    @pl.when(w < EXTRA)
    def _():
        pltpu.sync_copy(src_hbm.at[pl.ds((row_start + BASE) * CH, CH)],
                        src_v.at[pl.ds(BASE * CH, CH)])

    _dst_slab(ei_hbm.at[1], dst_v, row_start, w)

    pltpu.sync_copy(y_hbm.at[pl.ds(sid * RPT, RPT)],
                    y_sh.at[pl.ds(sid * RPT, RPT)])
    pltpu.sync_copy(zeros_hbm, acc_sh.at[pl.ds(sid * RPT, RPT)])
    plsc.subcore_barrier()

    def src_idx(j):
        return src_v.at[pl.ds(j * CH, CH)]

    def dst_idx(j):
        return dst_v.at[pl.ds(j * CH, CH)]

    bufs = (buf0, buf1, buf2)
    gsems = (gsem0, gsem1, gsem2)
    ssems = (ssem0, ssem1, ssem2)

    pltpu.async_copy(y_sh.at[src_idx(0)], buf0, gsem0)
    pltpu.async_copy(y_sh.at[src_idx(1)], buf1, gsem1)

    def trip(i, _):
        j0 = i * 3
        for b in range(3):
            j = j0 + b
            buf, gsem, ssem = bufs[b], gsems[b], ssems[b]
            b2 = (b + 2) % 3
            pltpu.make_async_copy(y_sh.at[src_idx(j)], buf, gsem).wait()
            if b == 0:
                @pl.when(i >= 1)
                def _():
                    pltpu.make_async_copy(
                        bufs[b2], acc_sh.at[dst_idx(j)], ssems[b2]).wait()
            else:
                pltpu.make_async_copy(
                    bufs[b2], acc_sh.at[dst_idx(j)], ssems[b2]).wait()
            pltpu.async_copy(buf, acc_sh.at[dst_idx(j)], ssem, add=True)

            @pl.when(j + 2 < ncap)
            def _():
                pltpu.async_copy(y_sh.at[src_idx(j + 2)], bufs[b2], gsems[b2])

        return ()

    lax.fori_loop(0, BASE // 3, trip, ())

    pltpu.make_async_copy(buf2, acc_sh.at[dst_idx(BASE - 1)], ssem2).wait()

    @pl.when(ncap > BASE)
    def _():
        pltpu.make_async_copy(y_sh.at[src_idx(BASE)], bufs[BASE % 3],
                              gsems[BASE % 3]).wait()
        pltpu.sync_copy(bufs[BASE % 3], acc_sh.at[dst_idx(BASE)], add=True)

    plsc.subcore_barrier()
    pltpu.sync_copy(acc_sh.at[pl.ds(sid * RPT, RPT)],
                    s_out.at[cid, pl.ds(sid * RPT, RPT)])


def _proj_body(x_ref, wg_ref, wps_ref, bps_ref, xw_ref, zsem_ref):
    x = x_ref[...]
    xw_ref[...] = jnp.dot(x, wg_ref[...], preferred_element_type=jnp.float32)
    zsem_ref[...] = (
        jnp.dot(x, wps_ref[...], preferred_element_type=jnp.float32)
        + bps_ref[...]
    )


def _proj(x, W_gcn, W_ps, b_ps):
    return pl.pallas_call(
        _proj_body,
        grid=(GRID,),
        in_specs=[
            pl.BlockSpec((BR, IN_DIM), lambda i: (i, 0)),
            pl.BlockSpec((IN_DIM, HID), lambda i: (0, 0)),
            pl.BlockSpec((IN_DIM, ALIGN), lambda i: (0, 0)),
            pl.BlockSpec((1, ALIGN), lambda i: (0, 0)),
        ],
        out_specs=[
            pl.BlockSpec((BR, HID), lambda i: (i, 0)),
            pl.BlockSpec((BR, ALIGN), lambda i: (i, 0)),
        ],
        out_shape=[
            jax.ShapeDtypeStruct((N, HID), jnp.float32),
            jax.ShapeDtypeStruct((N, ALIGN), jnp.float32),
        ],
    )(x, W_gcn, W_ps, b_ps)


def _scale_body(dp_ref, xw_ref, y_ref, dinv_ref):
    deg = dp_ref[0, :, 0:1] + dp_ref[1, :, 0:1] + 1.0
    dinv = lax.rsqrt(deg)
    y_ref[...] = dinv * xw_ref[...]
    dinv_ref[...] = jnp.broadcast_to(dinv, dinv_ref.shape)


def _scale(deg_parts, xw):
    return pl.pallas_call(
        _scale_body,
        grid=(GRID,),
        in_specs=[
            pl.BlockSpec((2, BR, 8), lambda i: (0, i, 0)),
            pl.BlockSpec((BR, HID), lambda i: (i, 0)),
        ],
        out_specs=[
            pl.BlockSpec((BR, HID), lambda i: (i, 0)),
            pl.BlockSpec((BR, 16), lambda i: (i, 0)),
        ],
        out_shape=[
            jax.ShapeDtypeStruct((SROWS, HID), jnp.float32),
            jax.ShapeDtypeStruct((N, 16), jnp.float32),
        ],
    )(deg_parts, xw)


def _head_body(s_ref, y_ref, dinv_ref, zsem_ref, bg_ref, wpt_ref, bpt_ref,
               wcls_ref, bcls_ref, logits_ref, an_ref, ztopo_ref):
    dinv = dinv_ref[:, 0:1]
    s_tot = s_ref[0] + s_ref[1] + y_ref[...]
    h = jnp.maximum(dinv * s_tot + bg_ref[...], 0.0)
    z_topo = (
        jnp.dot(h, wpt_ref[...], preferred_element_type=jnp.float32)
        + bpt_ref[...]
    )
    logits_ref[...] = (
        jnp.dot(z_topo, wcls_ref[...], preferred_element_type=jnp.float32)
        + bcls_ref[...]
    )
    diff = z_topo - zsem_ref[...]
    an_ref[...] = jnp.sqrt(jnp.sum(diff * diff, axis=1))
    ztopo_ref[...] = z_topo


def _heads(s_parts, y, dinv, z_sem, b_gcn, W_pt, b_pt, W_cls, b_cls):
    return pl.pallas_call(
        _head_body,
        grid=(GRID,),
        in_specs=[
            pl.BlockSpec((2, BR, HID), lambda i: (0, i, 0)),
            pl.BlockSpec((BR, HID), lambda i: (i, 0)),
            pl.BlockSpec((BR, 16), lambda i: (i, 0)),
            pl.BlockSpec((BR, ALIGN), lambda i: (i, 0)),
            pl.BlockSpec((1, HID), lambda i: (0, 0)),
            pl.BlockSpec((HID, ALIGN), lambda i: (0, 0)),
            pl.BlockSpec((1, ALIGN), lambda i: (0, 0)),
            pl.BlockSpec((ALIGN, NUM_CLASSES), lambda i: (0, 0)),
            pl.BlockSpec((1, NUM_CLASSES), lambda i: (0, 0)),
        ],
        out_specs=[
            pl.BlockSpec((BR, NUM_CLASSES), lambda i: (i, 0)),
            pl.BlockSpec((BR,), lambda i: (i,)),
            pl.BlockSpec((BR, ALIGN), lambda i: (i, 0)),
        ],
        out_shape=[
            jax.ShapeDtypeStruct((N, NUM_CLASSES), jnp.float32),
            jax.ShapeDtypeStruct((N,), jnp.float32),
            jax.ShapeDtypeStruct((N, ALIGN), jnp.float32),
        ],
    )(s_parts, y, dinv, z_sem, b_gcn, W_pt, b_pt, W_cls, b_cls)


def kernel(x, edge_index, W_gcn, b_gcn, W_pt, b_pt, W_ps, b_ps, W_cls, b_cls):

    zeros16 = jnp.zeros((RPT, 8), jnp.float32)
    zeros64 = jnp.zeros((RPT, HID), jnp.float32)
    ones16 = jnp.ones((CH, 8), jnp.float32)

    deg_parts = _deg_kernel(edge_index, zeros16, ones16)
    xw, z_sem = _proj(x, W_gcn, W_ps, b_ps.reshape(1, ALIGN))
    y, dinv = _scale(deg_parts, xw)
    s_parts = _msg_kernel(edge_index, y, zeros64)
    logits, anomaly, z_topo = _heads(
        s_parts, y, dinv, z_sem, b_gcn.reshape(1, HID), W_pt,
        b_pt.reshape(1, ALIGN), W_cls, b_cls.reshape(1, NUM_CLASSES))
    return (logits, anomaly, z_topo, z_sem)
